# Initial kernel scaffold; baseline (speedup 1.0000x reference)
import jax
import jax.numpy as jnp
from jax.experimental import pallas as pl


def kernel(weight, wgt_alpha):
    raise NotImplementedError("write your pallas kernel here")


# speedup vs baseline: 2.1362x; 2.1362x over previous
import functools

import numpy as np
import jax
import jax.numpy as jnp
from jax import lax
from jax.experimental import pallas as pl
from jax.experimental.pallas import tpu as pltpu
from jax.experimental.pallas import tpu_sc as plsc


def _build_lut() -> np.ndarray:
    value_a = [0.0] + [2.0 ** (-2 * i - 1) for i in range(3)]
    value_b = [0.0] + [2.0 ** (-2 * i - 2) for i in range(3)]
    vs = np.array(sorted({a + b for a in value_a for b in value_b}),
                  dtype=np.float32)
    levels = vs * (1.0 / vs.max())
    lev96 = np.round(np.float64(levels) * 96).astype(np.int64)
    mids = (lev96[:-1] + lev96[1:]) // 2
    tab_idx = np.array([(mids <= j).sum() for j in range(97)])
    tab = np.asarray(levels, np.float32)[tab_idx]
    return np.pad(tab, (0, _TAB_PAD - 97)).astype(np.float32)


_TAB_PAD = 112
_LUT = _build_lut()

_ROWS = 4096
_COLS = 4096
_N = _ROWS * _COLS

_RBLOCKS = 32
_BR = _ROWS // _RBLOCKS

_NC = 2
_NS = 16
_NW = _NC * _NS
_PER_W = _N // _NW
_CHUNK = 8192
_NBUF = 2
_NGROUP = _PER_W // (_CHUNK * _NBUF)
_L = 16


def _reduce_body(w_ref, out_ref):
    blk = w_ref[...]
    out_ref[0, 0, 0] = jnp.sum(blk)
    out_ref[0, 0, 1] = jnp.sum(blk * blk)


_reduce = pl.pallas_call(
    _reduce_body,
    grid=(_RBLOCKS,),
    in_specs=[pl.BlockSpec((_BR, _COLS), lambda i: (i, 0))],
    out_specs=pl.BlockSpec((1, 1, 2), lambda i: (i, 0, 0),
                           memory_space=pltpu.SMEM),
    out_shape=jax.ShapeDtypeStruct((_RBLOCKS, 1, 2), jnp.float32),
)


def _quant_body(w_hbm, sb_hbm, tab_hbm, out_hbm, inb0, inb1, outb0, outb1,
                tab_v, sb_v, *sems):
    inbufs = (inb0, inb1)
    outbufs = (outb0, outb1)
    wid = lax.axis_index("s") * _NC + lax.axis_index("c")
    base = wid * _PER_W

    pltpu.sync_copy(tab_hbm, tab_v)
    pltpu.sync_copy(sb_hbm, sb_v)
    s_vec = sb_v[pl.ds(0, _L)]
    b_vec = sb_v[pl.ds(_L, _L)]
    clamp = jnp.full((_L,), 96.0, jnp.float32)
    smask = jnp.full((_L,), np.int32(-2**31), jnp.int32)

    in_sems = sems[:_NBUF]
    out_sems = sems[_NBUF:]

    for b in range(_NBUF):
        pltpu.async_copy(w_hbm.at[pl.ds(base + b * _CHUNK, _CHUNK)],
                         inbufs[b], in_sems[b])

    def compute_chunk(b):
        cref = inbufs[b]
        oref = outbufs[b]

        def inner(i, carry):
            x = cref[pl.ds(i * _L, _L)]
            u = jnp.minimum(jnp.abs(x * s_vec + b_vec), clamp)
            idx = u.astype(jnp.int32)
            g = plsc.load_gather(tab_v, [idx])
            xi = lax.bitcast_convert_type(x, jnp.int32)
            gi = lax.bitcast_convert_type(g, jnp.int32)
            o = lax.bitcast_convert_type(gi | (xi & smask), jnp.float32)
            oref[pl.ds(i * _L, _L)] = o
            return carry

        lax.fori_loop(0, _CHUNK // _L, inner, 0, unroll=8)

    def group(gi, carry):
        for b in range(_NBUF):
            c = gi * _NBUF + b
            off = base + c * _CHUNK
            pltpu.make_async_copy(w_hbm.at[pl.ds(off, _CHUNK)],
                                  inbufs[b], in_sems[b]).wait()
            @pl.when(gi > 0)
            def _():
                pltpu.make_async_copy(
                    outbufs[b], out_hbm.at[pl.ds(off, _CHUNK)],
                    out_sems[b]).wait()

            compute_chunk(b)
            pltpu.async_copy(outbufs[b],
                             out_hbm.at[pl.ds(off, _CHUNK)], out_sems[b])

            @pl.when(gi + 1 < _NGROUP)
            def _():
                noff = base + (c + _NBUF) * _CHUNK
                pltpu.async_copy(w_hbm.at[pl.ds(noff, _CHUNK)],
                                 inbufs[b], in_sems[b])
        return carry

    lax.fori_loop(0, _NGROUP, group, 0)

    for b in range(_NBUF):
        pltpu.make_async_copy(
            outbufs[b], out_hbm.at[pl.ds(base + b * _CHUNK, _CHUNK)],
            out_sems[b]).wait()


_quantize = functools.partial(
    pl.kernel,
    out_type=jax.ShapeDtypeStruct((_N,), jnp.float32),
    mesh=plsc.VectorSubcoreMesh(core_axis_name="c", subcore_axis_name="s"),
    scratch_types=(
        [pltpu.VMEM((_CHUNK,), jnp.float32),
         pltpu.VMEM((_CHUNK,), jnp.float32),
         pltpu.VMEM((_CHUNK,), jnp.float32),
         pltpu.VMEM((_CHUNK,), jnp.float32),
         pltpu.VMEM((_TAB_PAD,), jnp.float32),
         pltpu.VMEM((2 * _L,), jnp.float32)]
        + [pltpu.SemaphoreType.DMA] * (2 * _NBUF)),
    compiler_params=pltpu.CompilerParams(needs_layout_passes=False),
)(_quant_body)


def kernel(weight, wgt_alpha):
    partials = _reduce(weight)
    total = jnp.sum(partials[:, 0, 0])
    total_sq = jnp.sum(partials[:, 0, 1])
    n = jnp.float32(_N)
    mean = total / n
    var = (total_sq - n * mean * mean) / (n - 1.0)
    std = jnp.sqrt(var)

    s = jnp.float32(96.0) / (std * wgt_alpha)
    b = -mean * s
    sb = jnp.concatenate([jnp.full((_L,), s), jnp.full((_L,), b)])
    tab = jnp.asarray(_LUT) * wgt_alpha

    flat = _quantize(weight.reshape(_N), sb, tab)
    return flat.reshape(_ROWS, _COLS)
